# baseline (speedup 1.0000x reference)
import jax
import jax.numpy as jnp
from jax.experimental import pallas as pl
from jax.experimental.pallas import tpu as pltpu


_TM = 512
_NBUF = 4


def _interp_pipeline(x_ref, s_hbm, o_ref, buf, sem):
    nsteps = s_hbm.shape[0] // _TM

    def copy_in(step, slot):
        return pltpu.make_async_copy(
            s_hbm.at[pl.ds(step * _TM, _TM), :],
            buf.at[slot],
            sem.at[slot],
        )

    for j in range(_NBUF):
        copy_in(j, j).start()

    def step_fn(i, carry):
        slot = jax.lax.rem(i, _NBUF)
        copy_in(i, slot).wait()
        o_ref[pl.ds(i * _TM, _TM), :] = jnp.dot(
            buf[slot], x_ref[...], preferred_element_type=jnp.float32)

        @pl.when(i + _NBUF < nsteps)
        def _():
            copy_in(i + _NBUF, slot).start()

        return carry

    jax.lax.fori_loop(0, nsteps, step_fn, 0)


def kernel(x_coarse, interp_matrix):
    m, k = interp_matrix.shape
    n = x_coarse.shape[1]
    return pl.pallas_call(
        _interp_pipeline,
        in_specs=[
            pl.BlockSpec(memory_space=pltpu.MemorySpace.VMEM),
            pl.BlockSpec(memory_space=pl.ANY),
        ],
        out_specs=pl.BlockSpec(memory_space=pltpu.MemorySpace.VMEM),
        out_shape=jax.ShapeDtypeStruct((m, n), jnp.float32),
        scratch_shapes=[
            pltpu.VMEM((_NBUF, _TM, 4096), jnp.float32),
            pltpu.SemaphoreType.DMA((_NBUF,)),
        ],
    )(x_coarse, interp_matrix)


# speedup vs baseline: 1.0556x; 1.0556x over previous
import jax
import jax.numpy as jnp
from jax.experimental import pallas as pl
from jax.experimental.pallas import tpu as pltpu


_TM = 512


def _interp_tile(s0_ref, s1_ref, x_ref, o_ref):
    kh = s0_ref.shape[1]
    o_ref[...] = (
        jnp.dot(s0_ref[...], x_ref[:kh, :], preferred_element_type=jnp.float32)
        + jnp.dot(s1_ref[...], x_ref[kh:, :], preferred_element_type=jnp.float32)
    )


def kernel(x_coarse, interp_matrix):
    m, k = interp_matrix.shape
    n = x_coarse.shape[1]
    kh = k // 2
    return pl.pallas_call(
        _interp_tile,
        grid=(m // _TM,),
        in_specs=[
            pl.BlockSpec((_TM, kh), lambda i: (i, 0)),
            pl.BlockSpec((_TM, kh), lambda i: (i, 1)),
            pl.BlockSpec(memory_space=pltpu.MemorySpace.VMEM),
        ],
        out_specs=pl.BlockSpec((_TM, n), lambda i: (i, 0)),
        out_shape=jax.ShapeDtypeStruct((m, n), jnp.float32),
    )(interp_matrix, interp_matrix, x_coarse)
